# initial kernel scaffold (unmeasured)
import jax
import jax.numpy as jnp
from jax import lax
from jax.experimental import pallas as pl
from jax.experimental.pallas import tpu as pltpu

N_DEV = 16


def kernel(x, w_mat):
    m_full, k_shard = x.shape
    k_full, n = w_mat.shape
    bm = m_full // N_DEV
    bk = k_full // N_DEV

    def body(x_hbm, w_hbm, out_ref, comm_ref, w_buf, send_sems, recv_sems,
             local_sem, w_sems):
        me = lax.axis_index("i")

        pltpu.make_async_copy(
            w_hbm.at[pl.ds(me * bk, bk), :], w_buf.at[0], w_sems.at[0]
        ).start()

        local_cp = pltpu.make_async_copy(
            x_hbm.at[pl.ds(me * bm, bm), :], comm_ref.at[me], local_sem
        )
        local_cp.start()

        for d in range(1, N_DEV):
            j = lax.rem(me - d + N_DEV, N_DEV)
            pltpu.make_async_remote_copy(
                src_ref=x_hbm.at[pl.ds(j * bm, bm), :],
                dst_ref=comm_ref.at[me],
                send_sem=send_sems.at[d],
                recv_sem=recv_sems.at[me],
                device_id=(j,),
                device_id_type=pl.DeviceIdType.MESH,
            ).start()

        for d in range(N_DEV):
            s = lax.rem(me + d, N_DEV)
            slot = d % 2
            pltpu.make_async_copy(
                w_hbm.at[pl.ds(s * bk, bk), :], w_buf.at[slot], w_sems.at[slot]
            ).wait()
            if d + 1 < N_DEV:
                s_nxt = lax.rem(me + d + 1, N_DEV)
                pltpu.make_async_copy(
                    w_hbm.at[pl.ds(s_nxt * bk, bk), :],
                    w_buf.at[(d + 1) % 2],
                    w_sems.at[(d + 1) % 2],
                ).start()
            if d == 0:
                local_cp.wait()
            else:
                pltpu.make_async_remote_copy(
                    src_ref=comm_ref.at[s],
                    dst_ref=comm_ref.at[s],
                    send_sem=send_sems.at[d],
                    recv_sem=recv_sems.at[s],
                    device_id=(s,),
                    device_id_type=pl.DeviceIdType.MESH,
                ).wait_recv()
            acc = jnp.dot(
                comm_ref[s], w_buf[slot], preferred_element_type=jnp.float32
            )
            if d == 0:
                out_ref[:, :] = acc
            else:
                out_ref[:, :] += acc

        for d in range(1, N_DEV):
            j = lax.rem(me - d + N_DEV, N_DEV)
            pltpu.make_async_remote_copy(
                src_ref=x_hbm.at[pl.ds(j * bm, bm), :],
                dst_ref=comm_ref.at[me],
                send_sem=send_sems.at[d],
                recv_sem=recv_sems.at[me],
                device_id=(j,),
                device_id_type=pl.DeviceIdType.MESH,
            ).wait_send()

    return pl.pallas_call(
        body,
        out_shape=jax.ShapeDtypeStruct((bm, n), jnp.float32),
        in_specs=[
            pl.BlockSpec(memory_space=pltpu.ANY),
            pl.BlockSpec(memory_space=pltpu.ANY),
        ],
        out_specs=pl.BlockSpec(memory_space=pltpu.VMEM),
        scratch_shapes=[
            pltpu.VMEM((N_DEV, bm, k_shard), jnp.float32),
            pltpu.VMEM((2, bk, n), jnp.float32),
            pltpu.SemaphoreType.DMA((N_DEV,)),
            pltpu.SemaphoreType.DMA((N_DEV,)),
            pltpu.SemaphoreType.DMA,
            pltpu.SemaphoreType.DMA((2,)),
        ],
        compiler_params=pltpu.CompilerParams(collective_id=0),
    )(x, w_mat)


# baseline (device time: 246512 ns/iter reference)
import jax
import jax.numpy as jnp
from jax import lax
from jax.experimental import pallas as pl
from jax.experimental.pallas import tpu as pltpu

N_DEV = 16


def kernel(x, w_mat):
    m_full, k_shard = x.shape
    k_full, n = w_mat.shape
    bm = m_full // N_DEV
    bk = k_full // N_DEV

    def body(x_hbm, w_hbm, out_ref, comm_ref, w_buf, send_sems, recv_sems,
             local_sem, w_sems):
        me = lax.axis_index("i")

        barrier_sem = pltpu.get_barrier_semaphore()
        for p in range(N_DEV):
            pl.when(p != me)(
                lambda p=p: pl.semaphore_signal(
                    barrier_sem, inc=1,
                    device_id=(p,), device_id_type=pl.DeviceIdType.MESH,
                )
            )
        pl.semaphore_wait(barrier_sem, N_DEV - 1)

        pltpu.make_async_copy(
            w_hbm.at[pl.ds(me * bk, bk), :], w_buf.at[0], w_sems.at[0]
        ).start()

        local_cp = pltpu.make_async_copy(
            x_hbm.at[pl.ds(me * bm, bm), :], comm_ref.at[me], local_sem
        )
        local_cp.start()

        for d in range(1, N_DEV):
            j = lax.rem(me - d + N_DEV, N_DEV)
            pltpu.make_async_remote_copy(
                src_ref=x_hbm.at[pl.ds(j * bm, bm), :],
                dst_ref=comm_ref.at[me],
                send_sem=send_sems.at[d],
                recv_sem=recv_sems.at[me],
                device_id=(j,),
                device_id_type=pl.DeviceIdType.MESH,
            ).start()

        for d in range(N_DEV):
            s = lax.rem(me + d, N_DEV)
            slot = d % 2
            pltpu.make_async_copy(
                w_hbm.at[pl.ds(s * bk, bk), :], w_buf.at[slot], w_sems.at[slot]
            ).wait()
            if d + 1 < N_DEV:
                s_nxt = lax.rem(me + d + 1, N_DEV)
                pltpu.make_async_copy(
                    w_hbm.at[pl.ds(s_nxt * bk, bk), :],
                    w_buf.at[(d + 1) % 2],
                    w_sems.at[(d + 1) % 2],
                ).start()
            if d == 0:
                local_cp.wait()
            else:
                pltpu.make_async_remote_copy(
                    src_ref=comm_ref.at[s],
                    dst_ref=comm_ref.at[s],
                    send_sem=send_sems.at[d],
                    recv_sem=recv_sems.at[s],
                    device_id=(s,),
                    device_id_type=pl.DeviceIdType.MESH,
                ).wait_recv()
            acc = jnp.dot(
                comm_ref[s], w_buf[slot], preferred_element_type=jnp.float32
            )
            if d == 0:
                out_ref[:, :] = acc
            else:
                out_ref[:, :] += acc

        for d in range(1, N_DEV):
            j = lax.rem(me - d + N_DEV, N_DEV)
            pltpu.make_async_remote_copy(
                src_ref=x_hbm.at[pl.ds(j * bm, bm), :],
                dst_ref=comm_ref.at[me],
                send_sem=send_sems.at[d],
                recv_sem=recv_sems.at[me],
                device_id=(j,),
                device_id_type=pl.DeviceIdType.MESH,
            ).wait_send()

    return pl.pallas_call(
        body,
        out_shape=jax.ShapeDtypeStruct((bm, n), jnp.float32),
        in_specs=[
            pl.BlockSpec(memory_space=pl.ANY),
            pl.BlockSpec(memory_space=pl.ANY),
        ],
        out_specs=pl.BlockSpec(memory_space=pltpu.VMEM),
        scratch_shapes=[
            pltpu.VMEM((N_DEV, bm, k_shard), jnp.float32),
            pltpu.VMEM((2, bk, n), jnp.float32),
            pltpu.SemaphoreType.DMA((N_DEV,)),
            pltpu.SemaphoreType.DMA((N_DEV,)),
            pltpu.SemaphoreType.DMA,
            pltpu.SemaphoreType.DMA((2,)),
        ],
        compiler_params=pltpu.CompilerParams(collective_id=0),
    )(x, w_mat)


# device time: 122422 ns/iter; 2.0136x vs baseline; 2.0136x over previous
import numpy as np

import jax
import jax.numpy as jnp
from jax import lax
from jax.experimental import pallas as pl
from jax.experimental.pallas import tpu as pltpu

N_DEV = 16

_POS = ((0, 0), (1, 0), (1, 1), (0, 1))


def _order_for(me: int) -> list[int]:
    pm, qm = me // 4, me % 4

    def key(s: int):
        ps, qs = s // 4, s % 4
        zd = abs(ps - pm)
        md = abs(_POS[qs][0] - _POS[qm][0]) + abs(_POS[qs][1] - _POS[qm][1])
        return (zd, md, (qs - qm) % 4)

    return sorted(range(N_DEV), key=key)


_ORDER = np.array([_order_for(me) for me in range(N_DEV)], np.int32)


def kernel(x, w_mat):
    m_full, k_shard = x.shape
    k_full, n = w_mat.shape
    bm = m_full // N_DEV
    bk = k_full // N_DEV

    xb = x.astype(jnp.bfloat16)
    order = jnp.asarray(_ORDER)

    def body(x_hbm, w_hbm, order, out_ref, comm_ref, w_buf, send_sems,
             recv_sems, local_sem, w_sems):
        me = lax.axis_index("i")

        pltpu.make_async_copy(
            w_hbm.at[pl.ds(me * bk, bk), :], w_buf.at[0], w_sems.at[0]
        ).start()
        local_cp = pltpu.make_async_copy(
            x_hbm.at[pl.ds(me * bm, bm), :], comm_ref.at[me], local_sem
        )
        local_cp.start()

        barrier_sem = pltpu.get_barrier_semaphore()
        for q in range(N_DEV):
            pl.when(q != me)(
                lambda q=q: pl.semaphore_signal(
                    barrier_sem, inc=1,
                    device_id=(q,), device_id_type=pl.DeviceIdType.MESH,
                )
            )
        pl.semaphore_wait(barrier_sem, N_DEV - 1)

        for dd in range(N_DEV - 1, 0, -1):
            j = order[me, dd]
            pltpu.make_async_remote_copy(
                src_ref=x_hbm.at[pl.ds(j * bm, bm), :],
                dst_ref=comm_ref.at[me],
                send_sem=send_sems.at[dd],
                recv_sem=recv_sems.at[me],
                device_id=(j,),
                device_id_type=pl.DeviceIdType.MESH,
            ).start()

        for d in range(N_DEV):
            s = order[me, d]
            slot = d % 2
            pltpu.make_async_copy(
                w_hbm.at[pl.ds(s * bk, bk), :], w_buf.at[slot], w_sems.at[slot]
            ).wait()
            if d + 1 < N_DEV:
                s_nxt = order[me, d + 1]
                pltpu.make_async_copy(
                    w_hbm.at[pl.ds(s_nxt * bk, bk), :],
                    w_buf.at[(d + 1) % 2],
                    w_sems.at[(d + 1) % 2],
                ).start()
            if d == 0:
                local_cp.wait()
            else:
                pltpu.make_async_remote_copy(
                    src_ref=comm_ref.at[s],
                    dst_ref=comm_ref.at[s],
                    send_sem=send_sems.at[d],
                    recv_sem=recv_sems.at[s],
                    device_id=(s,),
                    device_id_type=pl.DeviceIdType.MESH,
                ).wait_recv()
            acc = jnp.dot(
                comm_ref[s], w_buf[slot], preferred_element_type=jnp.float32
            )
            if d == 0:
                out_ref[:, :] = acc
            else:
                out_ref[:, :] += acc

        for dd in range(1, N_DEV):
            j = order[me, dd]
            pltpu.make_async_remote_copy(
                src_ref=x_hbm.at[pl.ds(j * bm, bm), :],
                dst_ref=comm_ref.at[me],
                send_sem=send_sems.at[dd],
                recv_sem=recv_sems.at[me],
                device_id=(j,),
                device_id_type=pl.DeviceIdType.MESH,
            ).wait_send()

    return pl.pallas_call(
        body,
        out_shape=jax.ShapeDtypeStruct((bm, n), jnp.float32),
        in_specs=[
            pl.BlockSpec(memory_space=pl.ANY),
            pl.BlockSpec(memory_space=pl.ANY),
            pl.BlockSpec(memory_space=pltpu.MemorySpace.SMEM),
        ],
        out_specs=pl.BlockSpec(memory_space=pltpu.VMEM),
        scratch_shapes=[
            pltpu.VMEM((N_DEV, bm, k_shard), jnp.bfloat16),
            pltpu.VMEM((2, bk, n), jnp.float32),
            pltpu.SemaphoreType.DMA((N_DEV,)),
            pltpu.SemaphoreType.DMA((N_DEV,)),
            pltpu.SemaphoreType.DMA,
            pltpu.SemaphoreType.DMA((2,)),
        ],
        compiler_params=pltpu.CompilerParams(
            collective_id=0, vmem_limit_bytes=100 * 1024 * 1024
        ),
    )(xb, w_mat, order)


# device time: 113063 ns/iter; 2.1803x vs baseline; 1.0828x over previous
import numpy as np

import jax
import jax.numpy as jnp
from jax import lax
from jax.experimental import pallas as pl
from jax.experimental.pallas import tpu as pltpu

N_DEV = 16

_POS = ((0, 0), (1, 0), (1, 1), (0, 1))


def _order_for(me: int) -> list[int]:
    pm, qm = me // 4, me % 4

    def key(s: int):
        ps, qs = s // 4, s % 4
        zd = abs(ps - pm)
        md = abs(_POS[qs][0] - _POS[qm][0]) + abs(_POS[qs][1] - _POS[qm][1])
        return (zd, md, (qs - qm) % 4)

    return sorted(range(N_DEV), key=key)


_ORDER = np.array([_order_for(me) for me in range(N_DEV)], np.int32)


def kernel(x, w_mat):
    m_full, k_shard = x.shape
    k_full, n = w_mat.shape
    bm = m_full // N_DEV
    bk = k_full // N_DEV

    order = jnp.asarray(_ORDER)

    def body(x_hbm, w_hbm, order, out_ref, x_bf, stage, comm_ref, w_buf,
             send_sems, recv_sems, stage_sems, w_sems):
        me = lax.axis_index("i")

        pltpu.make_async_copy(
            w_hbm.at[pl.ds(me * bk, bk), :], w_buf.at[0], w_sems.at[0]
        ).start()

        def blk(i):
            return me if i == 0 else order[me, N_DEV - i]

        def stage_dma(i, slot):
            return pltpu.make_async_copy(
                x_hbm.at[pl.ds(blk(i) * bm, bm), :],
                stage.at[slot],
                stage_sems.at[slot],
            )

        stage_dma(0, 0).start()
        stage_dma(1, 1).start()

        barrier_sem = pltpu.get_barrier_semaphore()
        for q in range(N_DEV):
            pl.when(q != me)(
                lambda q=q: pl.semaphore_signal(
                    barrier_sem, inc=1,
                    device_id=(q,), device_id_type=pl.DeviceIdType.MESH,
                )
            )

        stage_dma(0, 0).wait()
        j0 = blk(0)
        x_bf[pl.ds(j0 * bm, bm), :] = stage[0].astype(jnp.bfloat16)

        pl.semaphore_wait(barrier_sem, N_DEV - 1)

        for i in range(1, N_DEV):
            dd = N_DEV - i
            j = order[me, dd]
            stage_dma(i, i % 2).wait()
            if i + 1 < N_DEV:
                stage_dma(i + 1, (i + 1) % 2).start()
            x_bf[pl.ds(j * bm, bm), :] = stage[i % 2].astype(jnp.bfloat16)
            pltpu.make_async_remote_copy(
                src_ref=x_bf.at[pl.ds(j * bm, bm), :],
                dst_ref=comm_ref.at[me],
                send_sem=send_sems.at[dd],
                recv_sem=recv_sems.at[me],
                device_id=(j,),
                device_id_type=pl.DeviceIdType.MESH,
            ).start()

        for d in range(N_DEV):
            s = order[me, d]
            slot = d % 2
            pltpu.make_async_copy(
                w_hbm.at[pl.ds(s * bk, bk), :], w_buf.at[slot], w_sems.at[slot]
            ).wait()
            if d + 1 < N_DEV:
                s_nxt = order[me, d + 1]
                pltpu.make_async_copy(
                    w_hbm.at[pl.ds(s_nxt * bk, bk), :],
                    w_buf.at[(d + 1) % 2],
                    w_sems.at[(d + 1) % 2],
                ).start()
            if d == 0:
                chunk = x_bf[pl.ds(me * bm, bm), :]
            else:
                pltpu.make_async_remote_copy(
                    src_ref=comm_ref.at[s],
                    dst_ref=comm_ref.at[s],
                    send_sem=send_sems.at[d],
                    recv_sem=recv_sems.at[s],
                    device_id=(s,),
                    device_id_type=pl.DeviceIdType.MESH,
                ).wait_recv()
                chunk = comm_ref[s]
            acc = jnp.dot(
                chunk, w_buf[slot], preferred_element_type=jnp.float32
            )
            if d == 0:
                out_ref[:, :] = acc
            else:
                out_ref[:, :] += acc

        for dd in range(1, N_DEV):
            j = order[me, dd]
            pltpu.make_async_remote_copy(
                src_ref=x_bf.at[pl.ds(j * bm, bm), :],
                dst_ref=comm_ref.at[me],
                send_sem=send_sems.at[dd],
                recv_sem=recv_sems.at[me],
                device_id=(j,),
                device_id_type=pl.DeviceIdType.MESH,
            ).wait_send()

    return pl.pallas_call(
        body,
        out_shape=jax.ShapeDtypeStruct((bm, n), jnp.float32),
        in_specs=[
            pl.BlockSpec(memory_space=pl.ANY),
            pl.BlockSpec(memory_space=pl.ANY),
            pl.BlockSpec(memory_space=pltpu.MemorySpace.SMEM),
        ],
        out_specs=pl.BlockSpec(memory_space=pltpu.VMEM),
        scratch_shapes=[
            pltpu.VMEM((m_full, k_shard), jnp.bfloat16),
            pltpu.VMEM((2, bm, k_shard), jnp.float32),
            pltpu.VMEM((N_DEV, bm, k_shard), jnp.bfloat16),
            pltpu.VMEM((2, bk, n), jnp.float32),
            pltpu.SemaphoreType.DMA((N_DEV,)),
            pltpu.SemaphoreType.DMA((N_DEV,)),
            pltpu.SemaphoreType.DMA((2,)),
            pltpu.SemaphoreType.DMA((2,)),
        ],
        compiler_params=pltpu.CompilerParams(
            collective_id=0, vmem_limit_bytes=100 * 1024 * 1024
        ),
    )(x, w_mat, order)
